# Initial kernel scaffold; baseline (speedup 1.0000x reference)
import jax
import jax.numpy as jnp
from jax.experimental import pallas as pl


def kernel(feature, edge_index, W1, W2):
    raise NotImplementedError("write your pallas kernel here")


# speedup vs baseline: 24.3956x; 24.3956x over previous
import functools

import jax
import jax.numpy as jnp
from jax import lax
from jax.experimental import pallas as pl
from jax.experimental.pallas import tpu as pltpu
from jax.experimental.pallas import tpu_sc as plsc

N = 10000
E = 320000
D_IN = 128
D_HID = 16

NC = 2
NS = 16
NW = NC * NS
EPW = E // NW
CH = 100
G = 5
NCH = EPW // CH
NG = NCH // G
NP = 10240
RPT = NP // NS


def _seg_sum_partials(support, edges, fuse_relu_combine):
    mesh = plsc.VectorSubcoreMesh(core_axis_name="c", subcore_axis_name="s")

    @functools.partial(
        pl.kernel,
        mesh=mesh,
        compiler_params=pltpu.CompilerParams(use_tc_tiling_on_sc=False),
        out_type=jax.ShapeDtypeStruct((NC, NP, D_HID), jnp.float32),
        scratch_types=[
            pltpu.VMEM((NCH, CH), jnp.int32),
            pltpu.VMEM((NCH, CH), jnp.int32),
            ((pltpu.VMEM((CH, D_HID), jnp.float32),) * G,
             (pltpu.VMEM((CH, D_HID), jnp.float32),) * G),
            pltpu.VMEM((RPT, D_HID), jnp.float32),
            pltpu.VMEM((RPT, D_HID), jnp.float32),
            pltpu.VMEM_SHARED((NP, D_HID), jnp.float32),
            pltpu.VMEM_SHARED((NP, D_HID), jnp.float32),
            (pltpu.SemaphoreType.DMA,) * 2,
            (pltpu.SemaphoreType.DMA,) * 2,
        ],
    )
    def seg_kernel(sup_hbm, edge_hbm, out_hbm,
                   col_v, row_v, bufs, zbuf, pbuf, acc_sh, sup_sh, gsem, ssem):
        c = lax.axis_index("c")
        s = lax.axis_index("s")
        wid = c * NS + s

        def zero_body(i, _):
            zbuf[i, :] = jnp.zeros((D_HID,), jnp.float32)
            return 0
        lax.fori_loop(0, RPT, zero_body, 0)
        pltpu.sync_copy(zbuf, acc_sh.at[pl.ds(s * RPT, RPT)])

        if fuse_relu_combine:
            pltpu.sync_copy(sup_hbm.at[0, pl.ds(s * RPT, RPT)], zbuf)
            pltpu.sync_copy(sup_hbm.at[1, pl.ds(s * RPT, RPT)], pbuf)

            def comb_body(i, _):
                zbuf[i, :] = jnp.maximum(zbuf[i, :] + pbuf[i, :], 0.0)
                return 0
            lax.fori_loop(0, RPT, comb_body, 0)
            pltpu.sync_copy(zbuf, sup_sh.at[pl.ds(s * RPT, RPT)])
            dummy_src = sup_hbm.at[0, pl.ds(0, CH)]
        else:
            pltpu.sync_copy(sup_hbm.at[pl.ds(s * RPT, RPT)],
                            sup_sh.at[pl.ds(s * RPT, RPT)])
            dummy_src = sup_hbm.at[pl.ds(0, CH)]
        pltpu.sync_copy(edge_hbm.at[1, wid], col_v)
        pltpu.sync_copy(edge_hbm.at[0, wid], row_v)
        plsc.subcore_barrier()

        def drain(buf, sem):
            pltpu.make_async_copy(dummy_src, buf, sem).wait()

        def issue_group(g, p):
            for b in range(G):
                pltpu.async_copy(sup_sh.at[col_v.at[g * G + b]],
                                 bufs[p][b], gsem[p])

        def process_group(g, p, reissue):
            for b in range(G):
                drain(bufs[p][b], gsem[p])
                pltpu.async_copy(bufs[p][b], acc_sh.at[row_v.at[g * G + b]],
                                 ssem[p], add=True)
            for b in range(G):
                drain(bufs[p][b], ssem[p])
            if reissue:
                issue_group(g + 2, p)

        issue_group(0, 0)
        issue_group(1, 1)

        def body(t, _):
            process_group(t, 0, True)
            process_group(t + 1, 1, True)
            return 0
        lax.fori_loop(0, (NG - 2) // 2, lambda i, u: body(2 * i, u), 0,
                      unroll=False)
        process_group(NG - 2, 0, False)
        process_group(NG - 1, 1, False)

        plsc.subcore_barrier()
        pltpu.sync_copy(acc_sh.at[pl.ds(s * RPT, RPT)],
                        out_hbm.at[c, pl.ds(s * RPT, RPT)])

    return seg_kernel(support, edges)


def _matmul(x, w, block_m):
    _, K = x.shape
    Do = w.shape[1]

    def mm_kernel(x_ref, w_ref, o_hbm, o_vmem, sem):
        i = pl.program_id(0)
        o_vmem[...] = jnp.dot(x_ref[...], w_ref[...],
                              preferred_element_type=jnp.float32)
        pltpu.async_copy(o_vmem, o_hbm.at[pl.ds(i * block_m, block_m)],
                         sem).wait()

    return pl.pallas_call(
        mm_kernel,
        grid=(NP // block_m,),
        in_specs=[
            pl.BlockSpec((block_m, K), lambda i: (i, 0)),
            pl.BlockSpec((K, Do), lambda i: (0, 0)),
        ],
        out_specs=pl.BlockSpec(memory_space=pltpu.MemorySpace.HBM),
        out_shape=jax.ShapeDtypeStruct((NP, Do), jnp.float32),
        scratch_shapes=[
            pltpu.VMEM((block_m, Do), jnp.float32),
            pltpu.SemaphoreType.DMA,
        ],
    )(x, w)


def _combine_matmul(q, w, block_m):
    Do = w.shape[1]

    def cm_kernel(q_hbm, w_ref, o_ref, q0_v, q1_v, sem0, sem1):
        i = pl.program_id(0)
        c0 = pltpu.async_copy(q_hbm.at[0, pl.ds(i * block_m, block_m)],
                              q0_v, sem0)
        c1 = pltpu.async_copy(q_hbm.at[1, pl.ds(i * block_m, block_m)],
                              q1_v, sem1)
        c0.wait()
        c1.wait()
        o_ref[...] = jnp.dot(q0_v[...] + q1_v[...], w_ref[...],
                             preferred_element_type=jnp.float32)

    return pl.pallas_call(
        cm_kernel,
        grid=(NP // block_m,),
        in_specs=[
            pl.BlockSpec(memory_space=pltpu.MemorySpace.HBM),
            pl.BlockSpec((D_HID, Do), lambda i: (0, 0)),
        ],
        out_specs=pl.BlockSpec((block_m, Do), lambda i: (i, 0)),
        out_shape=jax.ShapeDtypeStruct((N, Do), jnp.float32),
        scratch_shapes=[
            pltpu.VMEM((block_m, D_HID), jnp.float32),
            pltpu.VMEM((block_m, D_HID), jnp.float32),
            pltpu.SemaphoreType.DMA,
            pltpu.SemaphoreType.DMA,
        ],
    )(q, w)


def kernel(feature, edge_index, W1, W2):
    edges = edge_index.reshape(2, NW, NCH, CH)
    support1 = _matmul(feature, W1, block_m=2048)
    p1 = _seg_sum_partials(support1, edges, fuse_relu_combine=False)
    p2 = _seg_sum_partials(p1, edges, fuse_relu_combine=True)
    return _combine_matmul(p2, W2, block_m=2048)


# speedup vs baseline: 28.2157x; 1.1566x over previous
import functools

import jax
import jax.numpy as jnp
from jax import lax
from jax.experimental import pallas as pl
from jax.experimental.pallas import tpu as pltpu
from jax.experimental.pallas import tpu_sc as plsc

N = 10000
E = 320000
D_IN = 128
D_HID = 16

NC = 2
NS = 16
NW = NC * NS
EPW = E // NW
CH = 100
G = 5
NCH = EPW // CH
NG = NCH // G
NP = 10240
RPT = NP // NS


def _seg_sum_partials(support, edges, fuse_relu_combine):
    mesh = plsc.VectorSubcoreMesh(core_axis_name="c", subcore_axis_name="s")

    @functools.partial(
        pl.kernel,
        mesh=mesh,
        compiler_params=pltpu.CompilerParams(use_tc_tiling_on_sc=False),
        out_type=jax.ShapeDtypeStruct((NC, NP // 8, 8 * D_HID), jnp.float32),
        scratch_types=[
            pltpu.VMEM((NCH, CH), jnp.int32),
            pltpu.VMEM((NCH, CH), jnp.int32),
            ((pltpu.VMEM((CH, D_HID), jnp.float32),) * G,
             (pltpu.VMEM((CH, D_HID), jnp.float32),) * G),
            pltpu.VMEM((RPT, D_HID), jnp.float32),
            pltpu.VMEM((RPT // 8, 8 * D_HID), jnp.float32),
            pltpu.VMEM((RPT // 8, 8 * D_HID), jnp.float32),
            pltpu.VMEM_SHARED((NP, D_HID), jnp.float32),
            pltpu.VMEM_SHARED((NP, D_HID), jnp.float32),
            (pltpu.SemaphoreType.DMA,) * 2,
            (pltpu.SemaphoreType.DMA,) * 2,
        ],
    )
    def seg_kernel(sup_hbm, edge_hbm, out_hbm,
                   col_v, row_v, bufs, zbuf, wbuf0, wbuf1, acc_sh, sup_sh,
                   gsem, ssem):
        c = lax.axis_index("c")
        s = lax.axis_index("s")
        wid = c * NS + s

        def zero_body(i, _):
            zbuf[i, :] = jnp.zeros((D_HID,), jnp.float32)
            return 0
        lax.fori_loop(0, RPT, zero_body, 0)
        pltpu.sync_copy(zbuf, acc_sh.at[pl.ds(s * RPT, RPT)])

        if fuse_relu_combine:
            pltpu.sync_copy(sup_hbm.at[0, pl.ds(s * RPT // 8, RPT // 8)],
                            wbuf0)
            pltpu.sync_copy(sup_hbm.at[1, pl.ds(s * RPT // 8, RPT // 8)],
                            wbuf1)

            def comb_body(r, _):
                for k in range(8):
                    sl = pl.ds(k * D_HID, D_HID)
                    zbuf[r * 8 + k, :] = jnp.maximum(
                        wbuf0[r, sl] + wbuf1[r, sl], 0.0)
                return 0
            lax.fori_loop(0, RPT // 8, comb_body, 0)
            pltpu.sync_copy(zbuf, sup_sh.at[pl.ds(s * RPT, RPT)])
        else:
            pltpu.sync_copy(sup_hbm.at[pl.ds(s * RPT, RPT)],
                            sup_sh.at[pl.ds(s * RPT, RPT)])
        dummy_src = edge_hbm.at[0, 0, pl.ds(0, CH * D_HID * 4 // (CH * 4))]
        pltpu.sync_copy(edge_hbm.at[1, wid], col_v)
        pltpu.sync_copy(edge_hbm.at[0, wid], row_v)
        plsc.subcore_barrier()

        def drain(buf, sem):
            pltpu.make_async_copy(dummy_src, buf, sem).wait()

        def issue_group(g, p):
            for b in range(G):
                pltpu.async_copy(sup_sh.at[col_v.at[g * G + b]],
                                 bufs[p][b], gsem[p])

        def process_group(g, p, reissue):
            for b in range(G):
                drain(bufs[p][b], gsem[p])
                pltpu.async_copy(bufs[p][b], acc_sh.at[row_v.at[g * G + b]],
                                 ssem[p], add=True)
            for b in range(G):
                drain(bufs[p][b], ssem[p])
            if reissue:
                issue_group(g + 2, p)

        issue_group(0, 0)
        issue_group(1, 1)

        def body(t, _):
            process_group(t, 0, True)
            process_group(t + 1, 1, True)
            return 0
        lax.fori_loop(0, (NG - 2) // 2, lambda i, u: body(2 * i, u), 0,
                      unroll=False)
        process_group(NG - 2, 0, False)
        process_group(NG - 1, 1, False)

        plsc.subcore_barrier()
        pltpu.sync_copy(acc_sh.at[pl.ds(s * RPT, RPT)], zbuf)

        def pack_body(r, _):
            for k in range(8):
                wbuf0[r, pl.ds(k * D_HID, D_HID)] = zbuf[r * 8 + k, :]
            return 0
        lax.fori_loop(0, RPT // 8, pack_body, 0)
        pltpu.sync_copy(wbuf0, out_hbm.at[c, pl.ds(s * RPT // 8, RPT // 8)])

    return seg_kernel(support, edges)


def _matmul(x, w, block_m):
    _, K = x.shape
    Do = w.shape[1]

    def mm_kernel(x_ref, w_ref, o_ref):
        o_ref[...] = jnp.dot(x_ref[...], w_ref[...],
                             preferred_element_type=jnp.float32)

    return pl.pallas_call(
        mm_kernel,
        grid=(N // block_m,),
        in_specs=[
            pl.BlockSpec((block_m, K), lambda i: (i, 0)),
            pl.BlockSpec((K, Do), lambda i: (0, 0)),
        ],
        out_specs=pl.BlockSpec((block_m, Do), lambda i: (i, 0)),
        out_shape=jax.ShapeDtypeStruct((NP, Do), jnp.float32),
    )(x, w)


def _combine_matmul(q, w, block_m):
    Do = w.shape[1] // 8

    def cm_kernel(q_ref, w_ref, o_ref):
        o_ref[...] = jnp.dot(q_ref[0] + q_ref[1], w_ref[...],
                             preferred_element_type=jnp.float32)

    bm8 = block_m // 8
    return pl.pallas_call(
        cm_kernel,
        grid=(NP // block_m,),
        in_specs=[
            pl.BlockSpec((NC, bm8, 8 * D_HID), lambda i: (0, i, 0)),
            pl.BlockSpec((8 * D_HID, 8 * Do), lambda i: (0, 0)),
        ],
        out_specs=pl.BlockSpec((bm8, 8 * Do), lambda i: (i, 0)),
        out_shape=jax.ShapeDtypeStruct((NP // 8, 8 * Do), jnp.float32),
    )(q, w)


def kernel(feature, edge_index, W1, W2):
    edges = edge_index.reshape(2, NW, NCH, CH)
    support1 = _matmul(feature, W1, block_m=2000)
    p1 = _seg_sum_partials(support1, edges, fuse_relu_combine=False)
    p2 = _seg_sum_partials(p1, edges, fuse_relu_combine=True)
    w2big = jnp.kron(jnp.eye(8, dtype=jnp.float32), W2)
    packed = _combine_matmul(p2, w2big, block_m=2048)
    return packed.reshape(NP, W2.shape[1])[:N]


# speedup vs baseline: 29.7670x; 1.0550x over previous
import functools

import jax
import jax.numpy as jnp
from jax import lax
from jax.experimental import pallas as pl
from jax.experimental.pallas import tpu as pltpu
from jax.experimental.pallas import tpu_sc as plsc

N = 10000
E = 320000
D_IN = 128
D_HID = 16

NC = 2
NS = 16
NW = NC * NS
EPW = E // NW
CH = 80
G = 5
NCH = EPW // CH
NG = NCH // G
NP = 10240
RPT = NP // NS


def _seg_sum_partials(support, edges, fuse_relu_combine):
    mesh = plsc.VectorSubcoreMesh(core_axis_name="c", subcore_axis_name="s")

    @functools.partial(
        pl.kernel,
        mesh=mesh,
        compiler_params=pltpu.CompilerParams(use_tc_tiling_on_sc=False,
                                            needs_layout_passes=False),
        out_type=jax.ShapeDtypeStruct((NC, NP // 8, 8 * D_HID), jnp.float32),
        scratch_types=[
            pltpu.VMEM((NCH, CH), jnp.int32),
            pltpu.VMEM((NCH, CH), jnp.int32),
            ((pltpu.VMEM((CH, D_HID), jnp.float32),) * G,
             (pltpu.VMEM((CH, D_HID), jnp.float32),) * G),
            pltpu.VMEM((RPT, D_HID), jnp.float32),
            pltpu.VMEM((RPT // 8, 8 * D_HID), jnp.float32),
            pltpu.VMEM((RPT // 8, 8 * D_HID), jnp.float32),
            pltpu.VMEM((D_HID, RPT), jnp.float32),
            pltpu.VMEM_SHARED((NP, D_HID), jnp.float32),
            pltpu.VMEM_SHARED((NP, D_HID), jnp.float32),
            (pltpu.SemaphoreType.DMA,) * 2,
            (pltpu.SemaphoreType.DMA,) * 2,
        ],
    )
    def seg_kernel(sup_hbm, edge_hbm, out_hbm,
                   col_v, row_v, bufs, zbuf, wbuf0, wbuf1, tbuf, acc_sh,
                   sup_sh, gsem, ssem):
        c = lax.axis_index("c")
        s = lax.axis_index("s")
        wid = c * NS + s

        def zero_body(i, _):
            zbuf[i, :] = jnp.zeros((D_HID,), jnp.float32)
            return 0
        lax.fori_loop(0, RPT, zero_body, 0)
        pltpu.sync_copy(zbuf, acc_sh.at[pl.ds(s * RPT, RPT)])

        if fuse_relu_combine:
            pltpu.sync_copy(sup_hbm.at[0, pl.ds(s * RPT // 8, RPT // 8)],
                            wbuf0)
            pltpu.sync_copy(sup_hbm.at[1, pl.ds(s * RPT // 8, RPT // 8)],
                            wbuf1)

            def comb_body(r, _):
                for k in range(8):
                    sl = pl.ds(k * D_HID, D_HID)
                    zbuf[r * 8 + k, :] = jnp.maximum(
                        wbuf0[r, sl] + wbuf1[r, sl], 0.0)
                return 0
            lax.fori_loop(0, RPT // 8, comb_body, 0)
            pltpu.sync_copy(zbuf, sup_sh.at[pl.ds(s * RPT, RPT)])
        else:
            pltpu.sync_copy(sup_hbm.at[:, pl.ds(s * RPT, RPT)], tbuf)
            lanes = lax.iota(jnp.int32, 16)

            def tr_body(i, _):
                zbuf[i, :] = plsc.load_gather(
                    tbuf, [lanes, jnp.full((16,), i, jnp.int32)])
                return 0
            lax.fori_loop(0, RPT, tr_body, 0)
            pltpu.sync_copy(zbuf, sup_sh.at[pl.ds(s * RPT, RPT)])
        dummy_src = edge_hbm.at[0, 0, pl.ds(0, CH * D_HID * 4 // (CH * 4))]
        pltpu.sync_copy(edge_hbm.at[1, wid], col_v)
        pltpu.sync_copy(edge_hbm.at[0, wid], row_v)
        plsc.subcore_barrier()

        def drain(buf, sem):
            pltpu.make_async_copy(dummy_src, buf, sem).wait()

        def issue_group(g, p):
            for b in range(G):
                pltpu.async_copy(sup_sh.at[col_v.at[g * G + b]],
                                 bufs[p][b], gsem[p])

        def process_group(g, p, reissue):
            for b in range(G):
                drain(bufs[p][b], gsem[p])
                pltpu.async_copy(bufs[p][b], acc_sh.at[row_v.at[g * G + b]],
                                 ssem[p], add=True)
            for b in range(G):
                drain(bufs[p][b], ssem[p])
            if reissue:
                issue_group(g + 2, p)

        issue_group(0, 0)
        issue_group(1, 1)

        def body(t, _):
            process_group(t, 0, True)
            process_group(t + 1, 1, True)
            return 0
        lax.fori_loop(0, (NG - 3) // 2, lambda i, u: body(2 * i, u), 0,
                      unroll=False)
        process_group(NG - 3, 0, True)
        process_group(NG - 2, 1, False)
        process_group(NG - 1, 0, False)

        plsc.subcore_barrier()
        pltpu.sync_copy(acc_sh.at[pl.ds(s * RPT, RPT)], zbuf)

        def pack_body(r, _):
            for k in range(8):
                wbuf0[r, pl.ds(k * D_HID, D_HID)] = zbuf[r * 8 + k, :]
            return 0
        lax.fori_loop(0, RPT // 8, pack_body, 0)
        pltpu.sync_copy(wbuf0, out_hbm.at[c, pl.ds(s * RPT // 8, RPT // 8)])

    return seg_kernel(support, edges)


def _matmul(x, w, block_m):
    _, K = x.shape
    Do = w.shape[1]

    def mm_kernel(x_ref, w_ref, o_ref):
        o_ref[...] = lax.dot_general(
            w_ref[...], x_ref[...],
            dimension_numbers=(((0,), (1,)), ((), ())),
            preferred_element_type=jnp.float32)

    return pl.pallas_call(
        mm_kernel,
        grid=(NP // block_m,),
        in_specs=[
            pl.BlockSpec((block_m, K), lambda i: (i, 0)),
            pl.BlockSpec((K, Do), lambda i: (0, 0)),
        ],
        out_specs=pl.BlockSpec((Do, block_m), lambda i: (0, i)),
        out_shape=jax.ShapeDtypeStruct((Do, NP), jnp.float32),
    )(x, w)


def _combine_matmul(q, w, block_m):
    Do = w.shape[1] // 8

    def cm_kernel(q_ref, w_ref, o_ref):
        o_ref[...] = jnp.dot(q_ref[0] + q_ref[1], w_ref[...],
                             preferred_element_type=jnp.float32)

    return pl.pallas_call(
        cm_kernel,
        grid=(1,),
        in_specs=[
            pl.BlockSpec((NC, NP // 8, 8 * D_HID), lambda i: (0, 0, 0)),
            pl.BlockSpec((8 * D_HID, 8 * Do), lambda i: (0, 0)),
        ],
        out_specs=pl.BlockSpec((NP // 8, 8 * Do), lambda i: (0, 0)),
        out_shape=jax.ShapeDtypeStruct((NP // 8, 8 * Do), jnp.float32),
    )(q, w)


def kernel(feature, edge_index, W1, W2):
    edges = edge_index.reshape(2, NW, NCH, CH)
    support1 = _matmul(feature, W1, block_m=2048)
    p1 = _seg_sum_partials(support1, edges, fuse_relu_combine=False)
    p2 = _seg_sum_partials(p1, edges, fuse_relu_combine=True)
    w2big = jnp.kron(jnp.eye(8, dtype=jnp.float32), W2)
    packed = _combine_matmul(p2, w2big, block_m=2048)
    return packed.reshape(NP, W2.shape[1])[:N]


# speedup vs baseline: 31.0077x; 1.0417x over previous
import functools

import jax
import jax.numpy as jnp
from jax import lax
from jax.experimental import pallas as pl
from jax.experimental.pallas import tpu as pltpu
from jax.experimental.pallas import tpu_sc as plsc

N = 10000
E = 320000
D_IN = 128
D_HID = 16

NC = 2
NS = 16
NW = NC * NS
EPW = E // NW
CH = 80
G = 5
NCH = EPW // CH
NG = NCH // G
NP = 10240
RPT = NP // NS


def _seg_sum_partials(support, edges, fuse_relu_combine):
    mesh = plsc.VectorSubcoreMesh(core_axis_name="c", subcore_axis_name="s")

    @functools.partial(
        pl.kernel,
        mesh=mesh,
        compiler_params=pltpu.CompilerParams(use_tc_tiling_on_sc=False,
                                            needs_layout_passes=False),
        out_type=jax.ShapeDtypeStruct((NC, NP // 8, 8 * D_HID), jnp.float32),
        scratch_types=[
            pltpu.VMEM((NCH, CH), jnp.int32),
            pltpu.VMEM((NCH, CH), jnp.int32),
            ((pltpu.VMEM((CH, D_HID), jnp.float32),) * G,
             (pltpu.VMEM((CH, D_HID), jnp.float32),) * G),
            pltpu.VMEM((RPT, D_HID), jnp.float32),
            pltpu.VMEM((RPT // 8, 8 * D_HID), jnp.float32),
            pltpu.VMEM((RPT // 8, 8 * D_HID), jnp.float32),
            pltpu.VMEM((D_HID, RPT + 1), jnp.float32),
            pltpu.VMEM_SHARED((NP, D_HID), jnp.float32),
            pltpu.VMEM_SHARED((NP, D_HID), jnp.float32),
            (pltpu.SemaphoreType.DMA,) * 2,
            (pltpu.SemaphoreType.DMA,) * 2,
        ],
    )
    def seg_kernel(sup_hbm, edge_hbm, out_hbm,
                   col_v, row_v, bufs, zbuf, wbuf0, wbuf1, tbuf, acc_sh,
                   sup_sh, gsem, ssem):
        c = lax.axis_index("c")
        s = lax.axis_index("s")
        wid = c * NS + s

        def zero_body(i, _):
            zbuf[i, :] = jnp.zeros((D_HID,), jnp.float32)
            return 0
        lax.fori_loop(0, RPT, zero_body, 0)
        pltpu.sync_copy(zbuf, acc_sh.at[pl.ds(s * RPT, RPT)])

        if fuse_relu_combine:
            pltpu.sync_copy(sup_hbm.at[0, pl.ds(s * RPT // 8, RPT // 8)],
                            wbuf0)
            pltpu.sync_copy(sup_hbm.at[1, pl.ds(s * RPT // 8, RPT // 8)],
                            wbuf1)

            def comb_body(r, _):
                for k in range(8):
                    sl = pl.ds(k * D_HID, D_HID)
                    zbuf[r * 8 + k, :] = jnp.maximum(
                        wbuf0[r, sl] + wbuf1[r, sl], 0.0)
                return 0
            lax.fori_loop(0, RPT // 8, comb_body, 0)
            pltpu.sync_copy(zbuf, sup_sh.at[pl.ds(s * RPT, RPT)])
        else:
            pltpu.sync_copy(sup_hbm.at[:, pl.ds(s * RPT, RPT)],
                            tbuf.at[:, pl.ds(0, RPT)])
            lanes = lax.iota(jnp.int32, 16)

            def tr_body(i, _):
                zbuf[i, :] = plsc.load_gather(
                    tbuf, [lanes, jnp.full((16,), i, jnp.int32)])
                return 0
            lax.fori_loop(0, RPT, tr_body, 0)
            pltpu.sync_copy(zbuf, sup_sh.at[pl.ds(s * RPT, RPT)])
        dummy_src = edge_hbm.at[0, 0, pl.ds(0, CH * D_HID * 4 // (CH * 4))]
        pltpu.sync_copy(edge_hbm.at[1, wid], col_v)
        pltpu.sync_copy(edge_hbm.at[0, wid], row_v)
        plsc.subcore_barrier()

        def drain(buf, sem):
            pltpu.make_async_copy(dummy_src, buf, sem).wait()

        def issue_group(g, p):
            for b in range(G):
                pltpu.async_copy(sup_sh.at[col_v.at[g * G + b]],
                                 bufs[p][b], gsem[p])

        def process_group(g, p, reissue):
            for b in range(G):
                drain(bufs[p][b], gsem[p])
                pltpu.async_copy(bufs[p][b], acc_sh.at[row_v.at[g * G + b]],
                                 ssem[p], add=True)
            for b in range(G):
                drain(bufs[p][b], ssem[p])
            if reissue:
                issue_group(g + 2, p)

        issue_group(0, 0)
        issue_group(1, 1)

        def body(t, _):
            process_group(t, 0, True)
            process_group(t + 1, 1, True)
            return 0
        lax.fori_loop(0, (NG - 3) // 2, lambda i, u: body(2 * i, u), 0,
                      unroll=False)
        process_group(NG - 3, 0, True)
        process_group(NG - 2, 1, False)
        process_group(NG - 1, 0, False)

        plsc.subcore_barrier()
        pltpu.sync_copy(acc_sh.at[pl.ds(s * RPT, RPT)], zbuf)

        def pack_body(r, _):
            for k in range(8):
                wbuf0[r, pl.ds(k * D_HID, D_HID)] = zbuf[r * 8 + k, :]
            return 0
        lax.fori_loop(0, RPT // 8, pack_body, 0)
        pltpu.sync_copy(wbuf0, out_hbm.at[c, pl.ds(s * RPT // 8, RPT // 8)])

    return seg_kernel(support, edges)


def _matmul(x, w, block_m):
    _, K = x.shape
    Do = w.shape[1]

    def mm_kernel(x_ref, w_ref, o_ref):
        o_ref[...] = lax.dot_general(
            w_ref[...], x_ref[...],
            dimension_numbers=(((0,), (1,)), ((), ())),
            preferred_element_type=jnp.float32)

    return pl.pallas_call(
        mm_kernel,
        grid=(NP // block_m,),
        in_specs=[
            pl.BlockSpec((block_m, K), lambda i: (i, 0)),
            pl.BlockSpec((K, Do), lambda i: (0, 0)),
        ],
        out_specs=pl.BlockSpec((Do, block_m), lambda i: (0, i)),
        out_shape=jax.ShapeDtypeStruct((Do, NP), jnp.float32),
    )(x, w)


def _combine_matmul(q, w, block_m):
    Do = w.shape[1] // 8

    def cm_kernel(q_ref, w_ref, o_ref):
        o_ref[...] = jnp.dot(q_ref[0] + q_ref[1], w_ref[...],
                             preferred_element_type=jnp.float32)

    return pl.pallas_call(
        cm_kernel,
        grid=(1,),
        in_specs=[
            pl.BlockSpec((NC, NP // 8, 8 * D_HID), lambda i: (0, 0, 0)),
            pl.BlockSpec((8 * D_HID, 8 * Do), lambda i: (0, 0)),
        ],
        out_specs=pl.BlockSpec((NP // 8, 8 * Do), lambda i: (0, 0)),
        out_shape=jax.ShapeDtypeStruct((NP // 8, 8 * Do), jnp.float32),
    )(q, w)


def kernel(feature, edge_index, W1, W2):
    edges = edge_index.reshape(2, NW, NCH, CH)
    support1 = _matmul(feature, W1, block_m=2048)
    p1 = _seg_sum_partials(support1, edges, fuse_relu_combine=False)
    p2 = _seg_sum_partials(p1, edges, fuse_relu_combine=True)
    w2big = jnp.kron(jnp.eye(8, dtype=jnp.float32), W2)
    packed = _combine_matmul(p2, w2big, block_m=2048)
    return packed.reshape(NP, W2.shape[1])[:N]
